# baseline (speedup 1.0000x reference)
import functools

import jax
import jax.numpy as jnp
from jax import lax
from jax.experimental import pallas as pl
from jax.experimental.pallas import tpu as pltpu
from jax.experimental.pallas import tpu_sc as plsc

_LANES = 16
_NC = 2
_NS = 16
_NW = _NC * _NS


_TC_ROWS = 1000


def _tc_body(n_rows, x_ref, w1_ref, b1_ref, wout_ref, bout_ref, t_ref, c_ref):
    hi = jax.lax.Precision.HIGHEST
    wv = jnp.dot(w1_ref[...], wout_ref[...], precision=hi,
                 preferred_element_type=jnp.float32)
    t_ref[...] = jnp.dot(x_ref[...], wv, precision=hi,
                         preferred_element_type=jnp.float32)
    cb = jnp.dot(b1_ref[...], wout_ref[...], precision=hi,
                 preferred_element_type=jnp.float32)
    c_ref[...] = cb * jnp.float32(n_rows) + bout_ref[...]


def _sc_gather_sum(t, src):
    n = t.shape[0]
    e = src.shape[0]
    epw = e // _NW
    iters = epw // _LANES
    unroll = 25
    assert iters % unroll == 0
    mesh = plsc.VectorSubcoreMesh(core_axis_name="c", subcore_axis_name="s")

    @functools.partial(
        pl.kernel,
        mesh=mesh,
        compiler_params=pltpu.CompilerParams(needs_layout_passes=False),
        out_type=jax.ShapeDtypeStruct((_NC * _LANES,), jnp.float32),
        scratch_types=[
            pltpu.VMEM((epw,), jnp.int32),
            pltpu.VMEM((n,), jnp.float32),
            pltpu.VMEM((_LANES,), jnp.float32),
            pltpu.VMEM((_NS * _LANES,), jnp.float32),
            pltpu.VMEM_SHARED((_NS * _LANES,), jnp.float32),
            pltpu.SemaphoreType.DMA,
            pltpu.SemaphoreType.DMA,
        ],
    )
    def k(t_hbm, src_hbm, out_hbm, idx_v, t_v, acc_v, all_v, part_sh,
          sem_t, sem_i):
        c = lax.axis_index("c")
        s = lax.axis_index("s")
        wid = s * _NC + c
        cp_t = pltpu.async_copy(t_hbm, t_v, sem_t)
        cp_i = pltpu.async_copy(src_hbm.at[pl.ds(wid * epw, epw)], idx_v, sem_i)
        cp_t.wait()
        cp_i.wait()

        def step(i, accs):
            base = i * (_LANES * unroll)
            vals = [
                plsc.load_gather(t_v, [idx_v[pl.ds(base + u * _LANES, _LANES)]])
                for u in range(unroll)
            ]
            return tuple(a + v for a, v in zip(accs, vals))

        zero = jnp.zeros((_LANES,), jnp.float32)
        accs = lax.fori_loop(0, iters // unroll, step, (zero,) * unroll)
        acc = accs[0]
        for a in accs[1:]:
            acc = acc + a
        acc_v[...] = acc
        pltpu.sync_copy(acc_v, part_sh.at[pl.ds(s * _LANES, _LANES)])
        plsc.subcore_barrier()

        @pl.when(s == 0)
        def _():
            pltpu.sync_copy(part_sh, all_v)
            tot = all_v[pl.ds(0, _LANES)]
            for j in range(1, _NS):
                tot = tot + all_v[pl.ds(j * _LANES, _LANES)]
            total = jnp.sum(tot)
            lane = lax.iota(jnp.int32, _LANES)
            acc_v[...] = jnp.where(lane == 0, total, jnp.float32(0.0))
            pltpu.sync_copy(acc_v, out_hbm.at[pl.ds(c * _LANES, _LANES)])

    return k(t, src)


def kernel(x, edge_index, W1, b1, W_out, b_out):
    src = edge_index[0]
    n, d = x.shape
    grid = n // _TC_ROWS
    t2, cst = pl.pallas_call(
        functools.partial(_tc_body, n),
        grid=(grid,),
        in_specs=[
            pl.BlockSpec((_TC_ROWS, d), lambda i: (i, 0)),
            pl.BlockSpec((d, W1.shape[1]), lambda i: (0, 0)),
            pl.BlockSpec((1, d), lambda i: (0, 0)),
            pl.BlockSpec((W_out.shape[0], 1), lambda i: (0, 0)),
            pl.BlockSpec((1, 1), lambda i: (0, 0)),
        ],
        out_specs=[
            pl.BlockSpec((_TC_ROWS, 1), lambda i: (i, 0)),
            pl.BlockSpec((1, 1), lambda i: (0, 0)),
        ],
        out_shape=[
            jax.ShapeDtypeStruct((n, 1), jnp.float32),
            jax.ShapeDtypeStruct((1, 1), jnp.float32),
        ],
    )(x, W1, b1.reshape(1, -1), W_out, b_out.reshape(1, 1))
    parts = _sc_gather_sum(t2.reshape(-1), src)
    out = parts[0] + parts[_LANES] + cst[0, 0]
    return out.reshape(1, 1)


# speedup vs baseline: 1.1019x; 1.1019x over previous
import functools

import jax
import jax.numpy as jnp
from jax import lax
from jax.experimental import pallas as pl
from jax.experimental.pallas import tpu as pltpu
from jax.experimental.pallas import tpu_sc as plsc

_LANES = 16
_NC = 2
_NS = 16
_NW = _NC * _NS


def _tc_body(x_ref, w1_ref, b1_ref, wout_ref, bout_ref, t_ref, c_ref):
    hi = jax.lax.Precision.HIGHEST
    wv = jnp.dot(w1_ref[...], wout_ref[...], precision=hi,
                 preferred_element_type=jnp.float32)
    t_ref[...] = jnp.dot(x_ref[...], wv, precision=hi,
                         preferred_element_type=jnp.float32)
    cb = jnp.dot(b1_ref[...], wout_ref[...], precision=hi,
                 preferred_element_type=jnp.float32)
    c_ref[...] = cb * jnp.float32(x_ref.shape[0]) + bout_ref[...]


def _sc_gather_sum(t, src):
    n = t.shape[0]
    e = src.shape[0]
    epw = e // _NW
    iters = epw // _LANES
    unroll = 25
    assert iters % unroll == 0
    mesh = plsc.VectorSubcoreMesh(core_axis_name="c", subcore_axis_name="s")

    @functools.partial(
        pl.kernel,
        mesh=mesh,
        compiler_params=pltpu.CompilerParams(needs_layout_passes=False),
        out_type=jax.ShapeDtypeStruct((_NC * _LANES,), jnp.float32),
        scratch_types=[
            pltpu.VMEM((epw,), jnp.int32),
            pltpu.VMEM((n,), jnp.float32),
            pltpu.VMEM((_LANES,), jnp.float32),
            pltpu.VMEM((_NS * _LANES,), jnp.float32),
            pltpu.VMEM_SHARED((_NS * _LANES,), jnp.float32),
            pltpu.SemaphoreType.DMA,
            pltpu.SemaphoreType.DMA,
        ],
    )
    def k(t_hbm, src_hbm, out_hbm, idx_v, t_v, acc_v, all_v, part_sh,
          sem_t, sem_i):
        c = lax.axis_index("c")
        s = lax.axis_index("s")
        wid = s * _NC + c
        cp_t = pltpu.async_copy(t_hbm, t_v, sem_t)
        cp_i = pltpu.async_copy(src_hbm.at[pl.ds(wid * epw, epw)], idx_v, sem_i)
        cp_t.wait()
        cp_i.wait()

        def step(i, accs):
            base = i * (_LANES * unroll)
            vals = [
                plsc.load_gather(t_v, [idx_v[pl.ds(base + u * _LANES, _LANES)]])
                for u in range(unroll)
            ]
            return tuple(a + v for a, v in zip(accs, vals))

        zero = jnp.zeros((_LANES,), jnp.float32)
        accs = lax.fori_loop(0, iters // unroll, step, (zero,) * unroll)
        acc = accs[0]
        for a in accs[1:]:
            acc = acc + a
        acc_v[...] = acc
        pltpu.sync_copy(acc_v, part_sh.at[pl.ds(s * _LANES, _LANES)])
        plsc.subcore_barrier()

        @pl.when(s == 0)
        def _():
            pltpu.sync_copy(part_sh, all_v)
            tot = all_v[pl.ds(0, _LANES)]
            for j in range(1, _NS):
                tot = tot + all_v[pl.ds(j * _LANES, _LANES)]
            total = jnp.sum(tot)
            lane = lax.iota(jnp.int32, _LANES)
            acc_v[...] = jnp.where(lane == 0, total, jnp.float32(0.0))
            pltpu.sync_copy(acc_v, out_hbm.at[pl.ds(c * _LANES, _LANES)])

    return k(t, src)


def kernel(x, edge_index, W1, b1, W_out, b_out):
    src = edge_index[0]
    t2, cst = pl.pallas_call(
        _tc_body,
        out_shape=[
            jax.ShapeDtypeStruct((x.shape[0], 1), jnp.float32),
            jax.ShapeDtypeStruct((1, 1), jnp.float32),
        ],
    )(x, W1, b1.reshape(1, -1), W_out, b_out.reshape(1, 1))
    parts = _sc_gather_sum(t2.reshape(-1), src)
    out = parts[0] + parts[_LANES] + cst[0, 0]
    return out.reshape(1, 1)
